# speedup vs baseline: 1.4604x; 1.4604x over previous
import jax
import jax.numpy as jnp
from jax.experimental import pallas as pl


def _add_body(x_ref, pos_ref, out_ref):
    out_ref[...] = x_ref[...] + pos_ref[...]


def kernel(x, pos_table):
    B, T, D = x.shape
    BT = 512
    grid = (B, T // BT)
    return pl.pallas_call(
        _add_body,
        grid=grid,
        in_specs=[
            pl.BlockSpec((1, BT, D), lambda b, t: (b, t, 0)),
            pl.BlockSpec((BT, D), lambda b, t: (t, 0)),
        ],
        out_specs=pl.BlockSpec((1, BT, D), lambda b, t: (b, t, 0)),
        out_shape=jax.ShapeDtypeStruct((B, T, D), x.dtype),
    )(x, pos_table)
